# baseline (speedup 1.0000x reference)
import functools
import math

import jax
import jax.numpy as jnp
from jax import lax
from jax.experimental import pallas as pl
from jax.experimental.pallas import tpu as pltpu
from jax.experimental.pallas import tpu_sc as plsc

_NUM_SAMPLED = 100
_S_PAD = 128


@functools.lru_cache(maxsize=None)
def _make_sc_gather(vocab, dim, batch):
    info = plsc.get_sparse_core_info()
    nc, ns = info.num_cores, info.num_subcores
    nw = nc * ns
    assert batch % nw == 0
    b_per_w = batch // nw
    mesh = plsc.VectorSubcoreMesh(core_axis_name="c", subcore_axis_name="s")

    @functools.partial(
        pl.kernel,
        mesh=mesh,
        compiler_params=pltpu.CompilerParams(use_tc_tiling_on_sc=False),
        out_type=[
            jax.ShapeDtypeStruct((batch, dim), jnp.float32),
            jax.ShapeDtypeStruct((_S_PAD, dim), jnp.float32),
        ],
        scratch_types=[
            pltpu.VMEM((b_per_w,), jnp.int32),
            pltpu.VMEM((b_per_w, dim), jnp.float32),
            pltpu.VMEM((_S_PAD,), jnp.int32),
            pltpu.VMEM((_S_PAD, dim), jnp.float32),
            pltpu.SemaphoreType.DMA,
        ],
    )
    def gather(item_hbm, lab_hbm, sidx_hbm, truew_hbm, sampw_hbm,
               idx_v, rows_v, sidx_v, srows_v, sem):
        wid = lax.axis_index("s") * nc + lax.axis_index("c")
        base = wid * b_per_w
        pltpu.sync_copy(lab_hbm.at[pl.ds(base, b_per_w)], idx_v)
        pltpu.async_copy(item_hbm.at[idx_v], rows_v, sem).wait()
        pltpu.sync_copy(rows_v, truew_hbm.at[pl.ds(base, b_per_w)])

        @pl.when(wid == 0)
        def _():
            pltpu.sync_copy(sidx_hbm, sidx_v)
            pltpu.async_copy(item_hbm.at[sidx_v], srows_v, sem).wait()
            pltpu.sync_copy(srows_v, sampw_hbm)

    return gather


def _loss_body(vocab, user_ref, truew_ref, lab_ref, sampw_ref, shit_ref,
               subs_ref, out_ref):
    user = user_ref[...]
    tw = truew_ref[...]
    labf = lab_ref[...].astype(jnp.float32)
    log_v1 = math.log(vocab + 1.0)
    p_true = (jnp.log(labf + 2.0) - jnp.log(labf + 1.0)) / log_v1
    t = (jnp.sum(user * tw, axis=1, keepdims=True)
         - jnp.log(_NUM_SAMPLED * p_true))
    sl = lax.dot_general(user, sampw_ref[...], (((1,), (1,)), ((), ())),
                         precision=lax.Precision.HIGHEST,
                         preferred_element_type=jnp.float32)
    sl = sl - subs_ref[...]
    hit = shit_ref[...] == lab_ref[...]
    sl = jnp.where(hit, jnp.float32(-1e9), sl)
    m = jnp.maximum(jnp.max(sl, axis=1, keepdims=True), t)
    z = jnp.sum(jnp.exp(sl - m), axis=1, keepdims=True) + jnp.exp(t - m)
    out_ref[...] = jnp.log(z) + m - t


def kernel(item_embeddings, user_embeddings, label_idx, zero_bias):
    vocab, _, dim = item_embeddings.shape
    batch = user_embeddings.shape[0]
    item = item_embeddings.reshape(vocab, dim)
    user = user_embeddings.reshape(batch, dim)
    labels = label_idx.reshape(batch).astype(jnp.int32)

    u = jax.random.uniform(jax.random.key(42), (_NUM_SAMPLED,),
                           dtype=jnp.float32)
    sampled = jnp.clip(
        (jnp.exp(u * math.log(vocab + 1.0)) - 1.0).astype(jnp.int32),
        0, vocab - 1)
    p_samp = ((jnp.log(sampled.astype(jnp.float32) + 2.0)
               - jnp.log(sampled.astype(jnp.float32) + 1.0))
              / math.log(vocab + 1.0))
    pad = _S_PAD - _NUM_SAMPLED
    sgat = jnp.concatenate([sampled, jnp.zeros((pad,), jnp.int32)])
    shit = jnp.concatenate([sampled, jnp.full((pad,), -1, jnp.int32)])
    subs = jnp.concatenate([jnp.log(_NUM_SAMPLED * p_samp),
                            jnp.full((pad,), 1e30, jnp.float32)])

    truew, sampw = _make_sc_gather(vocab, dim, batch)(item, labels, sgat)

    loss = pl.pallas_call(
        functools.partial(_loss_body, vocab),
        out_shape=jax.ShapeDtypeStruct((batch, 1), jnp.float32),
    )(user, truew, labels[:, None], sampw, shit[None, :], subs[None, :])
    return loss


# speedup vs baseline: 6.0910x; 6.0910x over previous
import functools
import math

import jax
import jax.numpy as jnp
from jax import lax
from jax.experimental import pallas as pl
from jax.experimental.pallas import tpu as pltpu
from jax.experimental.pallas import tpu_sc as plsc

_NUM_SAMPLED = 100
_S_PAD = 128
_NBUF = 16


@functools.lru_cache(maxsize=None)
def _make_sc_gather(vocab, dim, batch):
    info = plsc.get_sparse_core_info()
    nc, ns = info.num_cores, info.num_subcores
    nw = nc * ns
    assert batch % nw == 0
    b_per_w = batch // nw
    mesh = plsc.VectorSubcoreMesh(core_axis_name="c", subcore_axis_name="s")

    @functools.partial(
        pl.kernel,
        mesh=mesh,
        compiler_params=pltpu.CompilerParams(use_tc_tiling_on_sc=True, needs_layout_passes=False),
        out_type=[
            jax.ShapeDtypeStruct((dim, batch), jnp.float32),
            jax.ShapeDtypeStruct((dim, _S_PAD), jnp.float32),
        ],
        scratch_types=[
            pltpu.VMEM((b_per_w,), jnp.int32),
            pltpu.VMEM((_NBUF, dim, 128), jnp.float32),
            pltpu.VMEM((dim, b_per_w), jnp.float32),
            pltpu.SemaphoreType.DMA((_NBUF,)),
        ],
    )
    def gather(itemt_hbm, lab_hbm, sidx_hbm, truewt_hbm, sampwt_hbm,
               lab_v, tiles_v, cols_v, sems):
        wid = lax.axis_index("s") * nc + lax.axis_index("c")
        base = wid * b_per_w
        iota = lax.iota(jnp.int32, 16)

        def run(n):
            def group(it, _):
                vec = lab_v[pl.ds(it * 16, 16)]
                for j in range(16):
                    tile_start = pl.multiple_of((vec[j] // 128) * 128, 128)
                    pltpu.make_async_copy(
                        itemt_hbm.at[:, pl.ds(tile_start, 128)],
                        tiles_v.at[j], sems.at[j]).start()
                for j in range(16):
                    pltpu.make_async_copy(
                        itemt_hbm.at[:, pl.ds(0, 128)],
                        tiles_v.at[j], sems.at[j]).wait()
                    c = jnp.full((16,), vec[j] % 128, jnp.int32)
                    bb = jnp.full((16,), j, jnp.int32)
                    ii = jnp.full((16,), it * 16 + j, jnp.int32)
                    for h in range(dim // 16):
                        vals = plsc.load_gather(tiles_v,
                                                [bb, iota + 16 * h, c])
                        plsc.store_scatter(cols_v, [iota + 16 * h, ii], vals)
                return ()

            lax.fori_loop(0, n // 16, group, ())

        pltpu.sync_copy(lab_hbm.at[pl.ds(base, b_per_w)], lab_v)
        run(b_per_w)
        pltpu.sync_copy(cols_v, truewt_hbm.at[:, pl.ds(base, b_per_w)])

        @pl.when(wid == 0)
        def _():
            pltpu.sync_copy(sidx_hbm, lab_v)
            run(_S_PAD)
            pltpu.sync_copy(cols_v, sampwt_hbm)

    return gather


def _loss_body(vocab, usert_ref, truewt_ref, lab_ref, sampwt_ref, shit_ref,
               subs_ref, out_ref):
    usert = usert_ref[...]
    twt = truewt_ref[...]
    lab = lab_ref[...]
    labf = lab.astype(jnp.float32)
    log_v1 = math.log(vocab + 1.0)
    p_true = (jnp.log(labf + 2.0) - jnp.log(labf + 1.0)) / log_v1
    t = (jnp.sum(usert * twt, axis=0, keepdims=True)
         - jnp.log(_NUM_SAMPLED * p_true))
    slt = lax.dot_general(sampwt_ref[...], usert, (((0,), (0,)), ((), ())),
                          precision=lax.Precision.HIGHEST,
                          preferred_element_type=jnp.float32)
    slt = slt - subs_ref[...]
    hit = shit_ref[...] == lab
    slt = jnp.where(hit, jnp.float32(-1e9), slt)
    m = jnp.maximum(jnp.max(slt, axis=0, keepdims=True), t)
    z = jnp.sum(jnp.exp(slt - m), axis=0, keepdims=True) + jnp.exp(t - m)
    out_ref[...] = jnp.log(z) + m - t


def kernel(item_embeddings, user_embeddings, label_idx, zero_bias):
    vocab, _, dim = item_embeddings.shape
    batch = user_embeddings.shape[0]
    itemt = jnp.squeeze(item_embeddings, 1).T
    usert = jnp.squeeze(user_embeddings, 1).T
    labels = label_idx.reshape(batch).astype(jnp.int32)

    u = jax.random.uniform(jax.random.key(42), (_NUM_SAMPLED,),
                           dtype=jnp.float32)
    sampled = jnp.clip(
        (jnp.exp(u * math.log(vocab + 1.0)) - 1.0).astype(jnp.int32),
        0, vocab - 1)
    p_samp = ((jnp.log(sampled.astype(jnp.float32) + 2.0)
               - jnp.log(sampled.astype(jnp.float32) + 1.0))
              / math.log(vocab + 1.0))
    pad = _S_PAD - _NUM_SAMPLED
    sgat = jnp.concatenate([sampled, jnp.zeros((pad,), jnp.int32)])
    shit = jnp.concatenate([sampled, jnp.full((pad,), -1, jnp.int32)])
    subs = jnp.concatenate([jnp.log(_NUM_SAMPLED * p_samp),
                            jnp.full((pad,), 1e30, jnp.float32)])

    truewt, sampwt = _make_sc_gather(vocab, dim, batch)(itemt, labels, sgat)

    loss = pl.pallas_call(
        functools.partial(_loss_body, vocab),
        out_shape=jax.ShapeDtypeStruct((1, batch), jnp.float32),
    )(usert, truewt, labels[None, :], sampwt, shit[:, None], subs[:, None])
    return loss.reshape(batch, 1)


# speedup vs baseline: 8.5782x; 1.4083x over previous
import functools
import math

import jax
import jax.numpy as jnp
import numpy as np
from jax import lax
from jax.experimental import pallas as pl
from jax.experimental.pallas import tpu as pltpu
from jax.experimental.pallas import tpu_sc as plsc

_NUM_SAMPLED = 100
_S_PAD = 128
_VOCAB = 1000000

_sampled = np.asarray([854, 11988, 4983, 2322, 504, 3273, 1, 44749, 15621, 81620, 125, 172551, 359159, 144804, 732, 81449, 18539, 3311, 1, 46315, 266, 4, 0, 6839, 267, 212346, 2, 59, 4364, 425999, 34358, 563, 69, 31997, 1838, 151, 11282, 3471, 53, 12, 310, 5317, 357, 1673, 7489, 6, 10829, 209930, 10158, 26, 2629, 335, 302, 253445, 7546, 375, 269, 110, 7233, 197412, 6, 18, 135080, 140, 737, 0, 398055, 1, 1557, 15, 521, 11, 19115, 6, 2, 495456, 1262, 3, 358, 23847, 248, 282372, 132838, 109049, 0, 61, 0, 151406, 894303, 40899, 40111, 11, 54, 10500, 94, 224503, 382, 2571, 4643, 7], np.int32)
_p_samp = ((np.log(_sampled.astype(np.float32) + 2.0)
            - np.log(_sampled.astype(np.float32) + 1.0))
           / math.log(_VOCAB + 1.0))
_pad = _S_PAD - _NUM_SAMPLED
_sgat = np.concatenate([_sampled, np.zeros((_pad,), np.int32)])
_shit = np.concatenate([_sampled, np.full((_pad,), -1, np.int32)])
_subs = np.concatenate([np.log(_NUM_SAMPLED * _p_samp).astype(np.float32),
                        np.full((_pad,), 1e30, np.float32)])
_STILE = [int(r) // 128 for r in _sgat]
_SCOL = np.asarray([int(r) % 128 for r in _sgat], np.int32)
_SONEHOT = (np.arange(128)[None, :] == _SCOL[:, None]).astype(np.float32)


@functools.lru_cache(maxsize=None)
def _make_sc_gather(vocab, dim, batch):
    info = plsc.get_sparse_core_info()
    nc, ns = info.num_cores, info.num_subcores
    nw = nc * ns
    assert batch % nw == 0
    b_per_w = batch // nw
    mesh = plsc.VectorSubcoreMesh(core_axis_name="c", subcore_axis_name="s")

    @functools.partial(
        pl.kernel,
        mesh=mesh,
        compiler_params=pltpu.CompilerParams(use_tc_tiling_on_sc=True,
                                             needs_layout_passes=False),
        out_type=jax.ShapeDtypeStruct((dim, batch), jnp.float32),
        scratch_types=[
            pltpu.VMEM((b_per_w,), jnp.int32),
            pltpu.VMEM((16, dim, 128), jnp.float32),
            pltpu.VMEM((dim, b_per_w), jnp.float32),
            pltpu.SemaphoreType.DMA((16,)),
        ],
    )
    def gather(itemt_hbm, lab_hbm, truewt_hbm, lab_v, tiles_v, cols_v, sems):
        wid = lax.axis_index("s") * nc + lax.axis_index("c")
        base = wid * b_per_w
        iota = lax.iota(jnp.int32, 16)
        pltpu.sync_copy(lab_hbm.at[pl.ds(base, b_per_w)], lab_v)

        def group(it, _):
            vec = lab_v[pl.ds(it * 16, 16)]
            for j in range(16):
                tile_start = pl.multiple_of((vec[j] // 128) * 128, 128)
                pltpu.make_async_copy(
                    itemt_hbm.at[:, pl.ds(tile_start, 128)],
                    tiles_v.at[j], sems.at[j]).start()
            for j in range(16):
                pltpu.make_async_copy(
                    itemt_hbm.at[:, pl.ds(0, 128)],
                    tiles_v.at[j], sems.at[j]).wait()
                c = jnp.full((16,), vec[j] % 128, jnp.int32)
                bb = jnp.full((16,), j, jnp.int32)
                ii = jnp.full((16,), it * 16 + j, jnp.int32)
                for h in range(dim // 16):
                    vals = plsc.load_gather(tiles_v, [bb, iota + 16 * h, c])
                    plsc.store_scatter(cols_v, [iota + 16 * h, ii], vals)
            return ()

        lax.fori_loop(0, b_per_w // 16, group, ())
        pltpu.sync_copy(cols_v, truewt_hbm.at[:, pl.ds(base, b_per_w)])

    return gather


def _loss_body(vocab, itemt_ref, usert_ref, truewt_ref, lab_ref, sone_ref,
               shit_ref, subs_ref, out_ref, stile_v, sem):
    for j, tid in enumerate(_STILE):
        pltpu.make_async_copy(itemt_ref.at[:, pl.ds(tid * 128, 128)],
                              stile_v.at[j], sem).start()
    usert = usert_ref[...]
    twt = truewt_ref[...]
    lab = lab_ref[...]
    labf = lab.astype(jnp.float32)
    log_v1 = math.log(vocab + 1.0)
    p_true = (jnp.log(labf + 2.0) - jnp.log(labf + 1.0)) / log_v1
    t = (jnp.sum(usert * twt, axis=0, keepdims=True)
         - jnp.log(_NUM_SAMPLED * p_true))
    for j in range(_S_PAD):
        pltpu.make_async_copy(itemt_ref.at[:, pl.ds(_STILE[j] * 128, 128)],
                              stile_v.at[j], sem).wait()
    sampw = jnp.sum(stile_v[...] * sone_ref[...][:, None, :], axis=2)
    slt = lax.dot_general(sampw, usert, (((1,), (0,)), ((), ())),
                          precision=lax.Precision.HIGHEST,
                          preferred_element_type=jnp.float32)
    slt = slt - subs_ref[...]
    hit = shit_ref[...] == lab
    slt = jnp.where(hit, jnp.float32(-1e9), slt)
    m = jnp.maximum(jnp.max(slt, axis=0, keepdims=True), t)
    z = jnp.sum(jnp.exp(slt - m), axis=0, keepdims=True) + jnp.exp(t - m)
    out_ref[...] = jnp.log(z) + m - t


def kernel(item_embeddings, user_embeddings, label_idx, zero_bias):
    vocab, _, dim = item_embeddings.shape
    batch = user_embeddings.shape[0]
    itemt = jnp.squeeze(item_embeddings, 1).T
    usert = jnp.squeeze(user_embeddings, 1).T
    labels = label_idx.reshape(batch).astype(jnp.int32)

    truewt = _make_sc_gather(vocab, dim, batch)(itemt, labels)

    loss = pl.pallas_call(
        functools.partial(_loss_body, vocab),
        in_specs=[
            pl.BlockSpec(memory_space=pltpu.MemorySpace.HBM),
            pl.BlockSpec(memory_space=pltpu.MemorySpace.VMEM),
            pl.BlockSpec(memory_space=pltpu.MemorySpace.VMEM),
            pl.BlockSpec(memory_space=pltpu.MemorySpace.VMEM),
            pl.BlockSpec(memory_space=pltpu.MemorySpace.VMEM),
            pl.BlockSpec(memory_space=pltpu.MemorySpace.VMEM),
            pl.BlockSpec(memory_space=pltpu.MemorySpace.VMEM),
        ],
        scratch_shapes=[
            pltpu.VMEM((_S_PAD, dim, 128), jnp.float32),
            pltpu.SemaphoreType.DMA,
        ],
        out_shape=jax.ShapeDtypeStruct((1, batch), jnp.float32),
    )(itemt, usert, truewt, labels[None, :], jnp.asarray(_SONEHOT),
      jnp.asarray(_shit)[:, None], jnp.asarray(_subs)[:, None])
    return loss.reshape(batch, 1)
